# baseline (speedup 1.0000x reference)
import functools

import jax
import jax.numpy as jnp
from jax import lax
from jax.experimental import pallas as pl
from jax.experimental.pallas import tpu as pltpu
from jax.experimental.pallas import tpu_sc as plsc

N_NODES = 10000
N_PAD = 10240
N_EDGES = 320000
HID = 128
NC = 2
NS = 16
NW = NC * NS
NSLICE = 2
E_S = N_EDGES // NSLICE
EPW = E_S // NW
CH = 40
NCHUNK = EPW // CH
ROWS_PT = N_PAD // NS

_SENTINEL = 1.0e4


def _relu(v):
    return jnp.maximum(v, 0.0)


def _prep_body(x_ref, s_ref, npt_ref, w0, b0, w1, b1, w2, b2, out_ref):
    pts = x_ref[...]
    npt = npt_ref[...]
    npsq = jnp.sum(npt * npt, axis=0, keepdims=True)
    d2 = (jnp.sum(pts * pts, axis=1, keepdims=True)
          - 2.0 * jnp.dot(pts, npt, preferred_element_type=jnp.float32)
          + npsq)
    z = -d2
    z = z - jnp.max(z, axis=-1, keepdims=True)
    e = jnp.exp(z)
    sc = e / jnp.sum(e, axis=-1, keepdims=True)
    h = _relu(jnp.dot(s_ref[...], w0[...], preferred_element_type=jnp.float32) + b0[...])
    h = _relu(jnp.dot(h, w1[...], preferred_element_type=jnp.float32) + b1[...])
    emb = jnp.dot(h, w2[...], preferred_element_type=jnp.float32) + b2[...]
    lat = lax.dot_general(sc, emb, (((0,), (0,)), ((), ())),
                          preferred_element_type=jnp.float32)

    @pl.when(pl.program_id(0) == 0)
    def _():
        out_ref[...] = lat

    @pl.when(pl.program_id(0) != 0)
    def _():
        out_ref[...] += lat


def _prep(x, s, npt, enc):
    (w0, b0), (w1, b1), (w2, b2) = enc
    pb = 256
    grid = (x.shape[0] // pb,)
    const = lambda shape: pl.BlockSpec(shape, lambda i: (0,) * len(shape))
    return pl.pallas_call(
        _prep_body,
        grid=grid,
        in_specs=[
            pl.BlockSpec((pb, 3), lambda i: (i, 0)),
            pl.BlockSpec((pb, 8), lambda i: (i, 0)),
            const(npt.shape), const(w0.shape), const(b0.shape),
            const(w1.shape), const(b1.shape), const(w2.shape), const(b2.shape),
        ],
        out_specs=const((N_PAD, HID)),
        out_shape=jax.ShapeDtypeStruct((N_PAD, HID), jnp.float32),
    )(x, s, npt, w0, b0, w1, b1, w2, b2)


def _ab_body(lat_ref, wr, br, ws, out_a, out_b):
    lat = lat_ref[...]
    out_a[...] = jnp.dot(lat, wr[...], preferred_element_type=jnp.float32) + br[...]
    out_b[...] = jnp.dot(lat, ws[...], preferred_element_type=jnp.float32)


def _ab_tables(lat, w_first, b_first):
    nb = 2048
    grid = (N_PAD // nb,)
    const = lambda shape: pl.BlockSpec(shape, lambda i: (0,) * len(shape))
    return pl.pallas_call(
        _ab_body,
        grid=grid,
        in_specs=[
            pl.BlockSpec((nb, HID), lambda i: (i, 0)),
            const((HID, HID)), const((1, HID)), const((HID, HID)),
        ],
        out_specs=[pl.BlockSpec((nb, HID), lambda i: (i, 0)),
                   pl.BlockSpec((nb, HID), lambda i: (i, 0))],
        out_shape=[jax.ShapeDtypeStruct((N_PAD, HID), jnp.float32),
                   jax.ShapeDtypeStruct((N_PAD, HID), jnp.float32)],
    )(lat, w_first[:HID], b_first.reshape(1, HID), w_first[HID:])


def _gather_body(a_hbm, b_hbm, recv_hbm, send_hbm, outa_hbm, outb_hbm,
                 ridx, sidx, arows, brows, sga, sgb, swa, swb):
    wid = lax.axis_index("s") * NC + lax.axis_index("c")
    base = wid * EPW

    def off_of(i):
        return pl.multiple_of(base + i * CH, 8)

    def start(i, b):
        off = off_of(i)
        pltpu.sync_copy(recv_hbm.at[pl.ds(off, CH)], ridx.at[b])
        pltpu.sync_copy(send_hbm.at[pl.ds(off, CH)], sidx.at[b])
        pltpu.async_copy(a_hbm.at[ridx.at[b]], arows.at[b], sga)
        pltpu.async_copy(b_hbm.at[sidx.at[b]], brows.at[b], sgb)

    def wait_gather(b):
        pltpu.make_async_copy(a_hbm.at[ridx.at[b]], arows.at[b], sga).wait()
        pltpu.make_async_copy(b_hbm.at[sidx.at[b]], brows.at[b], sgb).wait()

    def start_wb(i, b):
        off = off_of(i)
        pltpu.async_copy(arows.at[b], outa_hbm.at[pl.ds(off, CH)], swa)
        pltpu.async_copy(brows.at[b], outb_hbm.at[pl.ds(off, CH)], swb)

    def wait_wb(i, b):
        off = off_of(i)
        pltpu.make_async_copy(arows.at[b], outa_hbm.at[pl.ds(off, CH)], swa).wait()
        pltpu.make_async_copy(brows.at[b], outb_hbm.at[pl.ds(off, CH)], swb).wait()

    start(0, 0)

    def chunk(i, carry):
        b = i % 2
        b2 = (i + 1) % 2
        wait_gather(b)
        start_wb(i, b)

        @pl.when(i + 1 < NCHUNK)
        def _():
            @pl.when(i >= 1)
            def _():
                wait_wb(i - 1, b2)

            start(i + 1, b2)

        return carry

    lax.fori_loop(0, NCHUNK, chunk, 0)
    wait_wb(NCHUNK - 2, (NCHUNK - 2) % 2)
    wait_wb(NCHUNK - 1, (NCHUNK - 1) % 2)


def _edge_gather(a_tab, b_tab, recv, send):
    mesh = plsc.VectorSubcoreMesh(core_axis_name="c", subcore_axis_name="s")
    f = pl.kernel(
        _gather_body,
        out_type=(jax.ShapeDtypeStruct((E_S, HID), jnp.float32),
                  jax.ShapeDtypeStruct((E_S, HID), jnp.float32)),
        mesh=mesh,
        scratch_types=[
            pltpu.VMEM((2, CH), jnp.int32),
            pltpu.VMEM((2, CH), jnp.int32),
            pltpu.VMEM((2, CH, HID), jnp.float32),
            pltpu.VMEM((2, CH, HID), jnp.float32),
            pltpu.SemaphoreType.DMA,
            pltpu.SemaphoreType.DMA,
            pltpu.SemaphoreType.DMA,
            pltpu.SemaphoreType.DMA,
        ],
    )
    return f(a_tab, b_tab, recv, send)


def _msg_body(a_ref, b_ref, w1, c1, w2, c2, out_ref):
    h = _relu(a_ref[...].astype(jnp.float32) + b_ref[...].astype(jnp.float32))
    h = _relu(jnp.dot(h, w1[...], preferred_element_type=jnp.float32) + c1[...])
    out_ref[...] = jnp.dot(h, w2[...], preferred_element_type=jnp.float32) + c2[...]


def _msg_mlp(ea, eb, msg_params):
    (_, _), (w1, c1), (w2, c2) = msg_params
    eb_blk = 3200
    grid = (E_S // eb_blk,)
    const = lambda shape: pl.BlockSpec(shape, lambda i: (0,) * len(shape))
    return pl.pallas_call(
        _msg_body,
        grid=grid,
        in_specs=[
            pl.BlockSpec((eb_blk, HID), lambda i: (i, 0)),
            pl.BlockSpec((eb_blk, HID), lambda i: (i, 0)),
            const((HID, HID)), const((1, HID)), const((HID, HID)), const((1, HID)),
        ],
        out_specs=pl.BlockSpec((eb_blk, HID), lambda i: (i, 0)),
        out_shape=jax.ShapeDtypeStruct((E_S, HID), jnp.float32),
    )(ea, eb, w1, c1.reshape(1, HID), w2, c2.reshape(1, HID))


def _scatter_body(msg_hbm, recv_hbm, zeros_hbm, out_hbm,
                  shared, idx, mrows, sli, slm, ssc):
    cid = lax.axis_index("c")
    sid = lax.axis_index("s")
    wid = sid * NC + cid
    base = wid * EPW
    row0 = sid * ROWS_PT
    pltpu.sync_copy(zeros_hbm.at[pl.ds(row0, ROWS_PT)],
                    shared.at[pl.ds(row0, ROWS_PT)])
    plsc.subcore_barrier()

    def off_of(i):
        return pl.multiple_of(base + i * CH, 8)

    def start_load(i, b):
        off = off_of(i)
        pltpu.async_copy(recv_hbm.at[pl.ds(off, CH)], idx.at[b], sli)
        pltpu.async_copy(msg_hbm.at[pl.ds(off, CH)], mrows.at[b], slm)

    def wait_load(i, b):
        off = off_of(i)
        pltpu.make_async_copy(recv_hbm.at[pl.ds(off, CH)], idx.at[b], sli).wait()
        pltpu.make_async_copy(msg_hbm.at[pl.ds(off, CH)], mrows.at[b], slm).wait()

    def wait_scatter(b):
        pltpu.make_async_copy(mrows.at[b], shared.at[idx.at[b]], ssc).wait()

    start_load(0, 0)

    def chunk(i, carry):
        b = i % 2
        b2 = (i + 1) % 2
        wait_load(i, b)
        pltpu.async_copy(mrows.at[b], shared.at[idx.at[b]], ssc, add=True)

        @pl.when(i + 1 < NCHUNK)
        def _():
            @pl.when(i >= 1)
            def _():
                wait_scatter(b2)

            start_load(i + 1, b2)

        return carry

    lax.fori_loop(0, NCHUNK, chunk, 0)
    wait_scatter((NCHUNK - 2) % 2)
    wait_scatter((NCHUNK - 1) % 2)
    plsc.subcore_barrier()
    pltpu.sync_copy(shared.at[pl.ds(row0, ROWS_PT)],
                    out_hbm.at[cid].at[pl.ds(row0, ROWS_PT)])


def _edge_scatter(messages, recv, zeros):
    mesh = plsc.VectorSubcoreMesh(core_axis_name="c", subcore_axis_name="s")
    f = pl.kernel(
        _scatter_body,
        out_type=jax.ShapeDtypeStruct((NC, N_PAD, HID), jnp.float32),
        mesh=mesh,
        scratch_types=[
            pltpu.VMEM_SHARED((N_PAD, HID), jnp.float32),
            pltpu.VMEM((2, CH), jnp.int32),
            pltpu.VMEM((2, CH, HID), jnp.float32),
            pltpu.SemaphoreType.DMA,
            pltpu.SemaphoreType.DMA,
            pltpu.SemaphoreType.DMA,
        ],
    )
    return f(messages, recv, zeros)


NPART = NSLICE * NC


def _node_body(lat_ref, p_ref, wl, wi, c0, w1, c1, w2, c2, out_ref):
    lat = lat_ref[...]
    p = p_ref[...]
    inbox = p[0] + p[1] + p[2] + p[3]
    h = _relu(jnp.dot(lat, wl[...], preferred_element_type=jnp.float32)
              + jnp.dot(inbox, wi[...], preferred_element_type=jnp.float32)
              + c0[...])
    h = _relu(jnp.dot(h, w1[...], preferred_element_type=jnp.float32) + c1[...])
    out_ref[...] = lat + jnp.dot(h, w2[...], preferred_element_type=jnp.float32) + c2[...]


def _node_update(lat, partials, node_params):
    (w0, c0), (w1, c1), (w2, c2) = node_params
    nb = 2048
    grid = (N_PAD // nb,)
    const = lambda shape: pl.BlockSpec(shape, lambda i: (0,) * len(shape))
    blk = pl.BlockSpec((nb, HID), lambda i: (i, 0))
    return pl.pallas_call(
        _node_body,
        grid=grid,
        in_specs=[
            blk,
            pl.BlockSpec((NPART, nb, HID), lambda i: (0, i, 0)),
            const((HID, HID)), const((HID, HID)), const((1, HID)),
            const((HID, HID)), const((1, HID)), const((HID, HID)), const((1, HID)),
        ],
        out_specs=blk,
        out_shape=jax.ShapeDtypeStruct((N_PAD, HID), jnp.float32),
    )(lat, partials, w0[:HID], w0[HID:], c0.reshape(1, HID),
      w1, c1.reshape(1, HID), w2, c2.reshape(1, HID))


def _dec_body(q_ref, npt_ref, lat_ref, wz, wq, c0, w1, c1, w2, c2, out_ref):
    pts = q_ref[...]
    npt = npt_ref[...]
    npsq = jnp.sum(npt * npt, axis=0, keepdims=True)
    d2 = (jnp.sum(pts * pts, axis=1, keepdims=True)
          - 2.0 * jnp.dot(pts, npt, preferred_element_type=jnp.float32)
          + npsq)
    z = -d2
    z = z - jnp.max(z, axis=-1, keepdims=True)
    e = jnp.exp(z)
    sc = e / jnp.sum(e, axis=-1, keepdims=True)
    zfeat = jnp.dot(sc, lat_ref[...], preferred_element_type=jnp.float32)
    h = _relu(jnp.dot(zfeat, wz[...], preferred_element_type=jnp.float32)
              + jnp.dot(pts, wq[...], preferred_element_type=jnp.float32)
              + c0[...])
    h = _relu(jnp.dot(h, w1[...], preferred_element_type=jnp.float32) + c1[...])
    out_ref[...] = jnp.dot(h, w2[...], preferred_element_type=jnp.float32) + c2[...]


def _decode(qpts, npt, lat, dec_params):
    (w0, c0), (w1, c1), (w2, c2) = dec_params
    qb = 256
    nq = qpts.shape[0]
    grid = (nq // qb,)
    const = lambda shape: pl.BlockSpec(shape, lambda i: (0,) * len(shape))
    return pl.pallas_call(
        _dec_body,
        grid=grid,
        in_specs=[
            pl.BlockSpec((qb, 3), lambda i: (i, 0)),
            const(npt.shape), const((N_PAD, HID)),
            const((HID, HID)), const((3, HID)), const((1, HID)),
            const((HID, HID)), const((1, HID)), const((HID, 8)), const((1, 8)),
        ],
        out_specs=pl.BlockSpec((qb, 8), lambda i: (i, 0)),
        out_shape=jax.ShapeDtypeStruct((nq, 8), jnp.float32),
    )(qpts, npt, lat, w0[:HID], w0[HID:], c0.reshape(1, HID),
      w1, c1.reshape(1, HID), w2, c2.reshape(1, 8))


def kernel(x, s, q, node_pos, senders, receivers, params):
    x2 = x[0]
    s2 = s[0]
    q2 = q[0]
    senders = senders.astype(jnp.int32)
    receivers = receivers.astype(jnp.int32)
    npt = jnp.full((3, N_PAD), _SENTINEL, jnp.float32)
    npt = npt.at[:, :N_NODES].set(node_pos.T)
    zeros = jnp.zeros((N_PAD, HID), jnp.float32)

    recv_s = [lax.slice_in_dim(receivers, k * E_S, (k + 1) * E_S) for k in range(NSLICE)]
    send_s = [lax.slice_in_dim(senders, k * E_S, (k + 1) * E_S) for k in range(NSLICE)]

    lat = _prep(x2, s2, npt, params["enc"])
    for bp in params["blocks"]:
        (w_first, b_first) = bp["msg"][0]
        a_tab, b_tab = _ab_tables(lat, w_first, b_first)
        parts = []
        for k in range(NSLICE):
            ea, eb = _edge_gather(a_tab, b_tab, recv_s[k], send_s[k])
            messages = _msg_mlp(ea, eb, bp["msg"])
            parts.append(_edge_scatter(messages, recv_s[k], zeros))
        partials = jnp.concatenate(parts, axis=0)
        lat = _node_update(lat, partials, bp["node"])
    out = _decode(q2, npt, lat, params["dec"])
    return out[None]


# speedup vs baseline: 1.3677x; 1.3677x over previous
import functools

import jax
import jax.numpy as jnp
from jax import lax
from jax.experimental import pallas as pl
from jax.experimental.pallas import tpu as pltpu
from jax.experimental.pallas import tpu_sc as plsc

N_NODES = 10000
N_PAD = 10240
N_EDGES = 320000
HID = 128
NC = 2
NS = 16
NW = NC * NS
NSLICE = 2
CH = 80
SLICE_CHUNKS = (62, 63)
SLICE_E = tuple(NW * CH * c for c in SLICE_CHUNKS)
ROWS_PT = N_PAD // NS

_SENTINEL = 1.0e4


def _relu(v):
    return jnp.maximum(v, 0.0)


def _prep_body(x_ref, s_ref, npt_ref, w0, b0, w1, b1, w2, b2, out_ref):
    pts = x_ref[...]
    npt = npt_ref[...]
    npsq = jnp.sum(npt * npt, axis=0, keepdims=True)
    d2 = (jnp.sum(pts * pts, axis=1, keepdims=True)
          - 2.0 * jnp.dot(pts, npt, preferred_element_type=jnp.float32)
          + npsq)
    z = -d2
    z = z - jnp.max(z, axis=-1, keepdims=True)
    e = jnp.exp(z)
    sc = e / jnp.sum(e, axis=-1, keepdims=True)
    h = _relu(jnp.dot(s_ref[...], w0[...], preferred_element_type=jnp.float32) + b0[...])
    h = _relu(jnp.dot(h, w1[...], preferred_element_type=jnp.float32) + b1[...])
    emb = jnp.dot(h, w2[...], preferred_element_type=jnp.float32) + b2[...]
    lat = lax.dot_general(sc, emb, (((0,), (0,)), ((), ())),
                          preferred_element_type=jnp.float32)

    @pl.when(pl.program_id(0) == 0)
    def _():
        out_ref[...] = lat

    @pl.when(pl.program_id(0) != 0)
    def _():
        out_ref[...] += lat


def _prep(x, s, npt, enc):
    (w0, b0), (w1, b1), (w2, b2) = enc
    pb = 256
    grid = (x.shape[0] // pb,)
    const = lambda shape: pl.BlockSpec(shape, lambda i: (0,) * len(shape))
    return pl.pallas_call(
        _prep_body,
        grid=grid,
        in_specs=[
            pl.BlockSpec((pb, 3), lambda i: (i, 0)),
            pl.BlockSpec((pb, 8), lambda i: (i, 0)),
            const(npt.shape), const(w0.shape), const(b0.shape),
            const(w1.shape), const(b1.shape), const(w2.shape), const(b2.shape),
        ],
        out_specs=const((N_PAD, HID)),
        out_shape=jax.ShapeDtypeStruct((N_PAD, HID), jnp.float32),
    )(x, s, npt, w0, b0, w1, b1, w2, b2)


def _ab_body(lat_ref, wr, br, ws, out_a, out_b):
    lat = lat_ref[...]
    out_a[...] = jnp.dot(lat, wr[...], preferred_element_type=jnp.float32) + br[...]
    out_b[...] = jnp.dot(lat, ws[...], preferred_element_type=jnp.float32)


def _ab_tables(lat, w_first, b_first):
    nb = 2048
    grid = (N_PAD // nb,)
    const = lambda shape: pl.BlockSpec(shape, lambda i: (0,) * len(shape))
    return pl.pallas_call(
        _ab_body,
        grid=grid,
        in_specs=[
            pl.BlockSpec((nb, HID), lambda i: (i, 0)),
            const((HID, HID)), const((1, HID)), const((HID, HID)),
        ],
        out_specs=[pl.BlockSpec((nb, HID), lambda i: (i, 0)),
                   pl.BlockSpec((nb, HID), lambda i: (i, 0))],
        out_shape=[jax.ShapeDtypeStruct((N_PAD, HID), jnp.float32),
                   jax.ShapeDtypeStruct((N_PAD, HID), jnp.float32)],
    )(lat, w_first[:HID], b_first.reshape(1, HID), w_first[HID:])


def _gather_body(nchunk, a_hbm, b_hbm, recv_hbm, send_hbm, outa_hbm, outb_hbm,
                 ridx, sidx, arows, brows, sga, sgb, swa, swb):
    wid = lax.axis_index("s") * NC + lax.axis_index("c")
    base = wid * (nchunk * CH)

    def off_of(i):
        return pl.multiple_of(base + i * CH, 8)

    def start(i, b):
        off = off_of(i)
        pltpu.sync_copy(recv_hbm.at[pl.ds(off, CH)], ridx.at[b])
        pltpu.sync_copy(send_hbm.at[pl.ds(off, CH)], sidx.at[b])
        pltpu.async_copy(a_hbm.at[ridx.at[b]], arows.at[b], sga)
        pltpu.async_copy(b_hbm.at[sidx.at[b]], brows.at[b], sgb)

    def wait_gather(b):
        pltpu.make_async_copy(a_hbm.at[ridx.at[b]], arows.at[b], sga).wait()
        pltpu.make_async_copy(b_hbm.at[sidx.at[b]], brows.at[b], sgb).wait()

    def start_wb(i, b):
        off = off_of(i)
        pltpu.async_copy(arows.at[b], outa_hbm.at[pl.ds(off, CH)], swa)
        pltpu.async_copy(brows.at[b], outb_hbm.at[pl.ds(off, CH)], swb)

    def wait_wb(i, b):
        off = off_of(i)
        pltpu.make_async_copy(arows.at[b], outa_hbm.at[pl.ds(off, CH)], swa).wait()
        pltpu.make_async_copy(brows.at[b], outb_hbm.at[pl.ds(off, CH)], swb).wait()

    start(0, 0)

    def chunk(i, carry):
        b = i % 2
        b2 = (i + 1) % 2
        wait_gather(b)
        start_wb(i, b)

        @pl.when(i + 1 < nchunk)
        def _():
            @pl.when(i >= 1)
            def _():
                wait_wb(i - 1, b2)

            start(i + 1, b2)

        return carry

    lax.fori_loop(0, nchunk, chunk, 0)
    wait_wb(nchunk - 2, (nchunk - 2) % 2)
    wait_wb(nchunk - 1, (nchunk - 1) % 2)


def _edge_gather(a_tab, b_tab, recv, send, nchunk):
    e_s = NW * CH * nchunk
    mesh = plsc.VectorSubcoreMesh(core_axis_name="c", subcore_axis_name="s")
    f = pl.kernel(
        functools.partial(_gather_body, nchunk),
        out_type=(jax.ShapeDtypeStruct((e_s, HID), jnp.float32),
                  jax.ShapeDtypeStruct((e_s, HID), jnp.float32)),
        mesh=mesh,
        scratch_types=[
            pltpu.VMEM((2, CH), jnp.int32),
            pltpu.VMEM((2, CH), jnp.int32),
            pltpu.VMEM((2, CH, HID), jnp.float32),
            pltpu.VMEM((2, CH, HID), jnp.float32),
            pltpu.SemaphoreType.DMA,
            pltpu.SemaphoreType.DMA,
            pltpu.SemaphoreType.DMA,
            pltpu.SemaphoreType.DMA,
        ],
    )
    return f(a_tab, b_tab, recv, send)


def _msg_body(a_ref, b_ref, w1, c1, w2, c2, out_ref):
    h = _relu(a_ref[...].astype(jnp.float32) + b_ref[...].astype(jnp.float32))
    h = _relu(jnp.dot(h, w1[...], preferred_element_type=jnp.float32) + c1[...])
    out_ref[...] = jnp.dot(h, w2[...], preferred_element_type=jnp.float32) + c2[...]


def _msg_mlp(ea, eb, msg_params):
    (_, _), (w1, c1), (w2, c2) = msg_params
    e_s = ea.shape[0]
    eb_blk = 2560
    grid = (e_s // eb_blk,)
    const = lambda shape: pl.BlockSpec(shape, lambda i: (0,) * len(shape))
    return pl.pallas_call(
        _msg_body,
        grid=grid,
        in_specs=[
            pl.BlockSpec((eb_blk, HID), lambda i: (i, 0)),
            pl.BlockSpec((eb_blk, HID), lambda i: (i, 0)),
            const((HID, HID)), const((1, HID)), const((HID, HID)), const((1, HID)),
        ],
        out_specs=pl.BlockSpec((eb_blk, HID), lambda i: (i, 0)),
        out_shape=jax.ShapeDtypeStruct((e_s, HID), jnp.float32),
    )(ea, eb, w1, c1.reshape(1, HID), w2, c2.reshape(1, HID))


def _scatter_body(nchunk, msg_hbm, recv_hbm, zeros_hbm, out_hbm,
                  shared, idx, mrows, sli, slm, ssc):
    cid = lax.axis_index("c")
    sid = lax.axis_index("s")
    wid = sid * NC + cid
    base = wid * (nchunk * CH)
    row0 = sid * ROWS_PT
    pltpu.sync_copy(zeros_hbm.at[pl.ds(row0, ROWS_PT)],
                    shared.at[pl.ds(row0, ROWS_PT)])
    plsc.subcore_barrier()

    def off_of(i):
        return pl.multiple_of(base + i * CH, 8)

    def start_load(i, b):
        off = off_of(i)
        pltpu.async_copy(recv_hbm.at[pl.ds(off, CH)], idx.at[b], sli)
        pltpu.async_copy(msg_hbm.at[pl.ds(off, CH)], mrows.at[b], slm)

    def wait_load(i, b):
        off = off_of(i)
        pltpu.make_async_copy(recv_hbm.at[pl.ds(off, CH)], idx.at[b], sli).wait()
        pltpu.make_async_copy(msg_hbm.at[pl.ds(off, CH)], mrows.at[b], slm).wait()

    def wait_scatter(b):
        pltpu.make_async_copy(mrows.at[b], shared.at[idx.at[b]], ssc).wait()

    start_load(0, 0)

    def chunk(i, carry):
        b = i % 2
        b2 = (i + 1) % 2
        wait_load(i, b)
        pltpu.async_copy(mrows.at[b], shared.at[idx.at[b]], ssc, add=True)

        @pl.when(i + 1 < nchunk)
        def _():
            @pl.when(i >= 1)
            def _():
                wait_scatter(b2)

            start_load(i + 1, b2)

        return carry

    lax.fori_loop(0, nchunk, chunk, 0)
    wait_scatter((nchunk - 2) % 2)
    wait_scatter((nchunk - 1) % 2)
    plsc.subcore_barrier()
    pltpu.sync_copy(shared.at[pl.ds(row0, ROWS_PT)],
                    out_hbm.at[cid].at[pl.ds(row0, ROWS_PT)])


def _edge_scatter(messages, recv, zeros, nchunk):
    mesh = plsc.VectorSubcoreMesh(core_axis_name="c", subcore_axis_name="s")
    f = pl.kernel(
        functools.partial(_scatter_body, nchunk),
        out_type=jax.ShapeDtypeStruct((NC, N_PAD, HID), jnp.float32),
        mesh=mesh,
        scratch_types=[
            pltpu.VMEM_SHARED((N_PAD, HID), jnp.float32),
            pltpu.VMEM((2, CH), jnp.int32),
            pltpu.VMEM((2, CH, HID), jnp.float32),
            pltpu.SemaphoreType.DMA,
            pltpu.SemaphoreType.DMA,
            pltpu.SemaphoreType.DMA,
        ],
    )
    return f(messages, recv, zeros)


NPART = NSLICE * NC


def _node_body(lat_ref, p_ref, wl, wi, c0, w1, c1, w2, c2, out_ref):
    lat = lat_ref[...]
    p = p_ref[...]
    inbox = p[0] + p[1] + p[2] + p[3]
    h = _relu(jnp.dot(lat, wl[...], preferred_element_type=jnp.float32)
              + jnp.dot(inbox, wi[...], preferred_element_type=jnp.float32)
              + c0[...])
    h = _relu(jnp.dot(h, w1[...], preferred_element_type=jnp.float32) + c1[...])
    out_ref[...] = lat + jnp.dot(h, w2[...], preferred_element_type=jnp.float32) + c2[...]


def _node_update(lat, partials, node_params):
    (w0, c0), (w1, c1), (w2, c2) = node_params
    nb = 2048
    grid = (N_PAD // nb,)
    const = lambda shape: pl.BlockSpec(shape, lambda i: (0,) * len(shape))
    blk = pl.BlockSpec((nb, HID), lambda i: (i, 0))
    return pl.pallas_call(
        _node_body,
        grid=grid,
        in_specs=[
            blk,
            pl.BlockSpec((NPART, nb, HID), lambda i: (0, i, 0)),
            const((HID, HID)), const((HID, HID)), const((1, HID)),
            const((HID, HID)), const((1, HID)), const((HID, HID)), const((1, HID)),
        ],
        out_specs=blk,
        out_shape=jax.ShapeDtypeStruct((N_PAD, HID), jnp.float32),
    )(lat, partials, w0[:HID], w0[HID:], c0.reshape(1, HID),
      w1, c1.reshape(1, HID), w2, c2.reshape(1, HID))


def _dec_body(q_ref, npt_ref, lat_ref, wz, wq, c0, w1, c1, w2, c2, out_ref):
    pts = q_ref[...]
    npt = npt_ref[...]
    npsq = jnp.sum(npt * npt, axis=0, keepdims=True)
    d2 = (jnp.sum(pts * pts, axis=1, keepdims=True)
          - 2.0 * jnp.dot(pts, npt, preferred_element_type=jnp.float32)
          + npsq)
    z = -d2
    z = z - jnp.max(z, axis=-1, keepdims=True)
    e = jnp.exp(z)
    sc = e / jnp.sum(e, axis=-1, keepdims=True)
    zfeat = jnp.dot(sc, lat_ref[...], preferred_element_type=jnp.float32)
    h = _relu(jnp.dot(zfeat, wz[...], preferred_element_type=jnp.float32)
              + jnp.dot(pts, wq[...], preferred_element_type=jnp.float32)
              + c0[...])
    h = _relu(jnp.dot(h, w1[...], preferred_element_type=jnp.float32) + c1[...])
    out_ref[...] = jnp.dot(h, w2[...], preferred_element_type=jnp.float32) + c2[...]


def _decode(qpts, npt, lat, dec_params):
    (w0, c0), (w1, c1), (w2, c2) = dec_params
    qb = 256
    nq = qpts.shape[0]
    grid = (nq // qb,)
    const = lambda shape: pl.BlockSpec(shape, lambda i: (0,) * len(shape))
    return pl.pallas_call(
        _dec_body,
        grid=grid,
        in_specs=[
            pl.BlockSpec((qb, 3), lambda i: (i, 0)),
            const(npt.shape), const((N_PAD, HID)),
            const((HID, HID)), const((3, HID)), const((1, HID)),
            const((HID, HID)), const((1, HID)), const((HID, 8)), const((1, 8)),
        ],
        out_specs=pl.BlockSpec((qb, 8), lambda i: (i, 0)),
        out_shape=jax.ShapeDtypeStruct((nq, 8), jnp.float32),
    )(qpts, npt, lat, w0[:HID], w0[HID:], c0.reshape(1, HID),
      w1, c1.reshape(1, HID), w2, c2.reshape(1, 8))


def kernel(x, s, q, node_pos, senders, receivers, params):
    x2 = x[0]
    s2 = s[0]
    q2 = q[0]
    senders = senders.astype(jnp.int32)
    receivers = receivers.astype(jnp.int32)
    npt = jnp.full((3, N_PAD), _SENTINEL, jnp.float32)
    npt = npt.at[:, :N_NODES].set(node_pos.T)
    zeros = jnp.zeros((N_PAD, HID), jnp.float32)

    bounds = [0]
    for e in SLICE_E:
        bounds.append(bounds[-1] + e)
    recv_s = [lax.slice_in_dim(receivers, bounds[k], bounds[k + 1]) for k in range(NSLICE)]
    send_s = [lax.slice_in_dim(senders, bounds[k], bounds[k + 1]) for k in range(NSLICE)]

    lat = _prep(x2, s2, npt, params["enc"])
    for bp in params["blocks"]:
        (w_first, b_first) = bp["msg"][0]
        a_tab, b_tab = _ab_tables(lat, w_first, b_first)
        parts = []
        for k in range(NSLICE):
            ea, eb = _edge_gather(a_tab, b_tab, recv_s[k], send_s[k], SLICE_CHUNKS[k])
            messages = _msg_mlp(ea, eb, bp["msg"])
            parts.append(_edge_scatter(messages, recv_s[k], zeros, SLICE_CHUNKS[k]))
        partials = jnp.concatenate(parts, axis=0)
        lat = _node_update(lat, partials, bp["node"])
    out = _decode(q2, npt, lat, params["dec"])
    return out[None]
